# baseline (speedup 1.0000x reference)
import jax
import jax.numpy as jnp
from jax.experimental import pallas as pl
from jax.experimental.pallas import tpu as pltpu

_ADJ_BLOCK_ROWS = 1024


def _body(root_ref, nf_ref, adj_hbm, feat_out_ref, adj_out_ref, sem):
    i = pl.program_id(0)
    cp = pltpu.make_async_copy(
        adj_hbm.at[pl.ds(i * _ADJ_BLOCK_ROWS, _ADJ_BLOCK_ROWS), :],
        adj_out_ref, sem)
    cp.start()

    @pl.when(i == 0)
    def _():
        x = nf_ref[...]
        rows = jax.lax.broadcasted_iota(jnp.int32, x.shape, 0)
        cols = jax.lax.broadcasted_iota(jnp.int32, x.shape, 1)
        mask = (cols < 2) & (rows != root_ref[0])
        feat_out_ref[...] = jnp.where(mask, jnp.float32(0.0), x)

    cp.wait()


def kernel(node_features, adjacency):
    num_nodes, feat = node_features.shape
    root = jax.random.randint(jax.random.key(1), (), 0, num_nodes).astype(jnp.int32)
    grid = (adjacency.shape[0] // _ADJ_BLOCK_ROWS,)
    out_features, adj_out = pl.pallas_call(
        _body,
        grid_spec=pltpu.PrefetchScalarGridSpec(
            num_scalar_prefetch=1,
            grid=grid,
            in_specs=[
                pl.BlockSpec((num_nodes, feat), lambda i, root: (0, 0)),
                pl.BlockSpec(memory_space=pl.MemorySpace.ANY),
            ],
            out_specs=[
                pl.BlockSpec((num_nodes, feat), lambda i, root: (0, 0)),
                pl.BlockSpec((_ADJ_BLOCK_ROWS, adjacency.shape[1]), lambda i, root: (i, 0)),
            ],
            scratch_shapes=[pltpu.SemaphoreType.DMA],
        ),
        out_shape=[
            jax.ShapeDtypeStruct((num_nodes, feat), node_features.dtype),
            jax.ShapeDtypeStruct(adjacency.shape, adjacency.dtype),
        ],
        compiler_params=pltpu.CompilerParams(
            dimension_semantics=("arbitrary",),
            vmem_limit_bytes=120 * 1024 * 1024,
        ),
    )(root.reshape((1,)), node_features, adjacency)
    return (out_features, adj_out)


# speedup vs baseline: 1.1258x; 1.1258x over previous
import jax
import jax.numpy as jnp
from jax.experimental import pallas as pl
from jax.experimental.pallas import tpu as pltpu

_ADJ_BLOCK_ROWS = 912


def _body(root_ref, nf_ref, adj_ref, feat_out_ref, adj_out_ref):
    adj_out_ref[...] = adj_ref[...]
    x = nf_ref[...]
    rows = jax.lax.broadcasted_iota(jnp.int32, x.shape, 0)
    cols = jax.lax.broadcasted_iota(jnp.int32, x.shape, 1)
    mask = (cols < 2) & (rows != root_ref[0])
    feat_out_ref[...] = jnp.where(mask, jnp.float32(0.0), x)


def kernel(node_features, adjacency):
    num_nodes, feat = node_features.shape
    root = jax.random.randint(jax.random.key(1), (), 0, num_nodes).astype(jnp.int32)
    grid = (pl.cdiv(adjacency.shape[0], _ADJ_BLOCK_ROWS),)
    out_features, adj_out = pl.pallas_call(
        _body,
        grid_spec=pltpu.PrefetchScalarGridSpec(
            num_scalar_prefetch=1,
            grid=grid,
            in_specs=[
                pl.BlockSpec((num_nodes, feat), lambda i, root: (0, 0)),
                pl.BlockSpec((_ADJ_BLOCK_ROWS, adjacency.shape[1]), lambda i, root: (i, 0)),
            ],
            out_specs=[
                pl.BlockSpec((num_nodes, feat), lambda i, root: (0, 0)),
                pl.BlockSpec((_ADJ_BLOCK_ROWS, adjacency.shape[1]), lambda i, root: (i, 0)),
            ],
        ),
        out_shape=[
            jax.ShapeDtypeStruct((num_nodes, feat), node_features.dtype),
            jax.ShapeDtypeStruct(adjacency.shape, adjacency.dtype),
        ],
        compiler_params=pltpu.CompilerParams(
            dimension_semantics=("arbitrary",),
            vmem_limit_bytes=120 * 1024 * 1024,
        ),
    )(root.reshape((1,)), node_features, adjacency)
    return (out_features, adj_out)
